# baseline (speedup 1.0000x reference)
import functools

import jax
import jax.numpy as jnp
import numpy as np
from jax import lax
from jax.experimental import pallas as pl
from jax.experimental.pallas import tpu as pltpu
from jax.experimental.pallas import tpu_sc as plsc

N = 10000
D = 128
AUG = 144
SLOPE = 0.1
NC, NS, L = 2, 16, 16
NW = NC * NS
K = 128
E_TOTAL = 320000 + N
EB = -(-E_TOTAL // (NW * K))
EPAD = EB * K * NW
NPAD = 10016
RPS = NPAD // NS
CHA = 27
NCHA = EB // CHA
CHB = 9
NCHB = EB // CHB
RBLK = 1000


def _emb_body(feat, w, b, a, emb16, s):
    emb = lax.dot_general(feat[...], w[...], (((1,), (1,)), ((), ())),
                          preferred_element_type=jnp.float32) + b[...]
    emb16[...] = emb.astype(jnp.bfloat16)
    s1 = lax.dot_general(emb, a[:D, :], (((1,), (0,)), ((), ())),
                         preferred_element_type=jnp.float32)
    s2 = lax.dot_general(emb, a[D:, :], (((1,), (0,)), ((), ())),
                         preferred_element_type=jnp.float32)
    s[...] = jnp.concatenate([s1, s2], axis=1)


_emb_call = pl.pallas_call(
    _emb_body,
    grid=(N // RBLK,),
    in_specs=[
        pl.BlockSpec((RBLK, D), lambda i: (i, 0)),
        pl.BlockSpec((D, D), lambda i: (0, 0)),
        pl.BlockSpec((1, D), lambda i: (0, 0)),
        pl.BlockSpec((2 * D, 1), lambda i: (0, 0)),
    ],
    out_specs=[
        pl.BlockSpec((RBLK, D), lambda i: (i, 0)),
        pl.BlockSpec((RBLK, 2), lambda i: (i, 0)),
    ],
    out_shape=[
        jax.ShapeDtypeStruct((N, D), jnp.bfloat16),
        jax.ShapeDtypeStruct((N, 2), jnp.float32),
    ],
)

_PERM = np.array(
    [32 * g + (k // 2 + (16 if k % 2 else 0)) for g in range(D // 32)
     for k in range(32)], dtype=np.int32)


_mesh = plsc.VectorSubcoreMesh(core_axis_name="c", subcore_axis_name="s")


@functools.partial(
    pl.kernel,
    out_type=[
        jax.ShapeDtypeStruct((NC, NPAD, AUG), jnp.float32),
        jax.ShapeDtypeStruct((NW, EB, K), jnp.float32),
    ],
    mesh=_mesh,
    compiler_params=pltpu.CompilerParams(needs_layout_passes=False,
                                         use_tc_tiling_on_sc=False),
    scratch_types=[
        pltpu.VMEM_SHARED((NPAD, AUG), jnp.float32),
        pltpu.SemaphoreType.DMA,
        pltpu.SemaphoreType.DMA,
        pltpu.SemaphoreType.DMA,
    ],
)
def _agg(src_hbm, dst_hbm, s_hbm, emb_hbm, zeros_hbm, out_hbm, w_hbm,
         acc_sh, sg0, sg1, ss0):
    c = lax.axis_index("c")
    sid = lax.axis_index("s")
    wid = sid * NC + c

    pltpu.sync_copy(zeros_hbm, acc_sh.at[pl.ds(sid * RPS, RPS)])
    plsc.subcore_barrier()

    def phase_a(s_v, sidx, didx, wbuf):
        pltpu.sync_copy(s_hbm, s_v)

        def chunk_body(ch, carry):
            pltpu.sync_copy(src_hbm.at[wid, pl.ds(ch * CHA, CHA)], sidx)
            pltpu.sync_copy(dst_hbm.at[wid, pl.ds(ch * CHA, CHA)], didx)

            def b_body(bj, carry2):
                for j in range(K // L):
                    sl = pl.ds(j * L, L)
                    srcv = sidx[bj, sl]
                    dstv = didx[bj, sl]
                    s1 = plsc.load_gather(s_v, [srcv * 2])
                    s2 = plsc.load_gather(s_v, [dstv * 2 + 1])
                    x = s1 + s2
                    wbuf[bj, pl.ds(j * L, L)] = jnp.exp(
                        jnp.maximum(x, x * SLOPE))
                return carry2

            lax.fori_loop(0, CHA, b_body, carry)
            pltpu.sync_copy(wbuf, w_hbm.at[wid, pl.ds(ch * CHA, CHA)])
            return carry

        lax.fori_loop(0, NCHA, chunk_body, 0)

    pl.run_scoped(phase_a,
                  pltpu.VMEM((2 * NPAD,), jnp.float32),
                  pltpu.VMEM((CHA, K), jnp.int32),
                  pltpu.VMEM((CHA, K), jnp.int32),
                  pltpu.VMEM((CHA, K), jnp.float32))

    def phase_b(sidx, didx, w_v, gb0, gb1, rows_v):
        gbuf = (gb0, gb1)
        gsem = (sg0, sg1)
        onehot = jnp.where(lax.iota(jnp.int32, L) == 0, 1.0, 0.0)

        def scale(gb, bj):
            def scale_body(i, carry3):
                wspl = plsc.load_gather(
                    w_v, [jnp.full((L,), bj, jnp.int32),
                          jnp.full((L,), i, jnp.int32)])
                for g in range(D // 32):
                    v = gb[i, pl.ds(g * L, L)]
                    vb = plsc.bitcast(v, jnp.bfloat16)
                    av, bv = plsc.unpack(
                        vb, format=plsc.PackFormat.INTERLEAVED)
                    rows_v[i, pl.ds(32 * g, L)] = av * wspl
                    rows_v[i, pl.ds(32 * g + L, L)] = bv * wspl
                rows_v[i, pl.ds(D, L)] = wspl * onehot
                return carry3

            lax.fori_loop(0, K, scale_body, 0)

        def chunk_body(ch, carry):
            pltpu.sync_copy(src_hbm.at[wid, pl.ds(ch * CHB, CHB)], sidx)
            pltpu.sync_copy(dst_hbm.at[wid, pl.ds(ch * CHB, CHB)], didx)
            pltpu.sync_copy(w_hbm.at[wid, pl.ds(ch * CHB, CHB)], w_v)
            pltpu.async_copy(emb_hbm.at[didx.at[0]], gbuf[0], gsem[0])
            for bj in range(CHB):
                p = bj % 2
                q = 1 - p
                pltpu.make_async_copy(
                    emb_hbm.at[didx.at[bj]], gbuf[p], gsem[p]).wait()
                if bj + 1 < CHB:
                    pltpu.async_copy(
                        emb_hbm.at[didx.at[bj + 1]], gbuf[q], gsem[q])
                if bj >= 1:
                    pltpu.make_async_copy(
                        rows_v, acc_sh.at[sidx.at[bj - 1]], ss0).wait()
                scale(gbuf[p], bj)
                pltpu.async_copy(
                    rows_v, acc_sh.at[sidx.at[bj]], ss0, add=True)
            pltpu.make_async_copy(
                rows_v, acc_sh.at[sidx.at[CHB - 1]], ss0).wait()
            return carry

        lax.fori_loop(0, NCHB, chunk_body, 0)

    pl.run_scoped(phase_b,
                  pltpu.VMEM((CHB, K), jnp.int32),
                  pltpu.VMEM((CHB, K), jnp.int32),
                  pltpu.VMEM((CHB, K), jnp.float32),
                  pltpu.VMEM((K, D // 2), jnp.float32),
                  pltpu.VMEM((K, D // 2), jnp.float32),
                  pltpu.VMEM((K, AUG), jnp.float32))

    plsc.subcore_barrier()
    pltpu.sync_copy(acc_sh.at[pl.ds(sid * RPS, RPS)],
                    out_hbm.at[c, pl.ds(sid * RPS, RPS)])


def _combine_body(p, o):
    tot = p[0] + p[1]
    rs = tot[:, D:D + 1]
    rs = jnp.where(rs == 0.0, 1.0, rs)
    o[...] = tot[:, :D] / rs


_combine_call = pl.pallas_call(
    _combine_body,
    grid=(N // RBLK,),
    in_specs=[pl.BlockSpec((NC, RBLK, AUG), lambda i: (0, i, 0))],
    out_specs=pl.BlockSpec((RBLK, D), lambda i: (i, 0)),
    out_shape=jax.ShapeDtypeStruct((N, D), jnp.float32),
)


def kernel(feat_table, nodes, edge_index, W, b, a):
    pad = EPAD - E_TOTAL
    nodes32 = nodes.astype(jnp.int32)
    src_all = jnp.concatenate(
        [edge_index[0], nodes32, jnp.full((pad,), N, jnp.int32)])
    dst_all = jnp.concatenate(
        [edge_index[1], nodes32, jnp.zeros((pad,), jnp.int32)])
    src_r = src_all.reshape(NW, EB, K)
    dst_r = dst_all.reshape(NW, EB, K)
    emb16, s = _emb_call(feat_table, W, b.reshape(1, D), a)
    packed = lax.bitcast_convert_type(
        emb16[:, _PERM].reshape(N, D // 2, 2), jnp.float32)
    s_pad = jnp.pad(s, ((0, NPAD - N), (0, 0))).reshape(-1)
    zeros = jnp.zeros((RPS, AUG), jnp.float32)
    partial, _ = _agg(src_r, dst_r, s_pad, packed, zeros)
    return _combine_call(partial)


# speedup vs baseline: 1.0813x; 1.0813x over previous
import functools

import jax
import jax.numpy as jnp
import numpy as np
from jax import lax
from jax.experimental import pallas as pl
from jax.experimental.pallas import tpu as pltpu
from jax.experimental.pallas import tpu_sc as plsc

N = 10000
D = 128
AUG = 144
SLOPE = 0.1
NC, NS, L = 2, 16, 16
NW = NC * NS
K = 128
E_TOTAL = 320000 + N
EB = -(-E_TOTAL // (NW * K))
EPAD = EB * K * NW
NPAD = 10016
RPS = NPAD // NS
CHA = 27
NCHA = EB // CHA
CHB = 9
NCHB = EB // CHB
RBLK = 1000


def _emb_body(feat, w, b, a, emb16, s):
    emb = lax.dot_general(feat[...], w[...], (((1,), (1,)), ((), ())),
                          preferred_element_type=jnp.float32) + b[...]
    emb16[...] = emb.astype(jnp.bfloat16)
    s1 = lax.dot_general(emb, a[:D, :], (((1,), (0,)), ((), ())),
                         preferred_element_type=jnp.float32)
    s2 = lax.dot_general(emb, a[D:, :], (((1,), (0,)), ((), ())),
                         preferred_element_type=jnp.float32)
    s[...] = jnp.concatenate([s1, s2], axis=1)


_emb_call = pl.pallas_call(
    _emb_body,
    grid=(N // RBLK,),
    in_specs=[
        pl.BlockSpec((RBLK, D), lambda i: (i, 0)),
        pl.BlockSpec((D, D), lambda i: (0, 0)),
        pl.BlockSpec((1, D), lambda i: (0, 0)),
        pl.BlockSpec((2 * D, 1), lambda i: (0, 0)),
    ],
    out_specs=[
        pl.BlockSpec((RBLK, D), lambda i: (i, 0)),
        pl.BlockSpec((RBLK, 2), lambda i: (i, 0)),
    ],
    out_shape=[
        jax.ShapeDtypeStruct((N, D), jnp.bfloat16),
        jax.ShapeDtypeStruct((N, 2), jnp.float32),
    ],
)

_PERM = np.array(
    [32 * g + (k // 2 + (16 if k % 2 else 0)) for g in range(D // 32)
     for k in range(32)], dtype=np.int32)


_mesh = plsc.VectorSubcoreMesh(core_axis_name="c", subcore_axis_name="s")


@functools.partial(
    pl.kernel,
    out_type=[
        jax.ShapeDtypeStruct((NC, NPAD, D), jnp.float32),
        jax.ShapeDtypeStruct((NW, EB, K), jnp.float32),
        jax.ShapeDtypeStruct((NW, NPAD), jnp.float32),
    ],
    mesh=_mesh,
    compiler_params=pltpu.CompilerParams(needs_layout_passes=False,
                                         use_tc_tiling_on_sc=False),
    scratch_types=[
        pltpu.VMEM_SHARED((NPAD, D), jnp.float32),
        pltpu.SemaphoreType.DMA,
        pltpu.SemaphoreType.DMA,
        pltpu.SemaphoreType.DMA,
    ],
)
def _agg(src_hbm, dst_hbm, s_hbm, emb_hbm, zeros_hbm, zrs_hbm, out_hbm,
         w_hbm, rs_hbm, acc_sh, sg0, sg1, ss0):
    c = lax.axis_index("c")
    sid = lax.axis_index("s")
    wid = sid * NC + c

    pltpu.sync_copy(zeros_hbm, acc_sh.at[pl.ds(sid * RPS, RPS)])
    plsc.subcore_barrier()

    def phase_a(s_v, rowsum_v, sidx, didx, wbuf):
        pltpu.sync_copy(s_hbm, s_v)
        pltpu.sync_copy(zrs_hbm, rowsum_v)

        def chunk_body(ch, carry):
            pltpu.sync_copy(src_hbm.at[wid, pl.ds(ch * CHA, CHA)], sidx)
            pltpu.sync_copy(dst_hbm.at[wid, pl.ds(ch * CHA, CHA)], didx)

            def b_body(bj, carry2):
                for j in range(K // L):
                    sl = pl.ds(j * L, L)
                    srcv = sidx[bj, sl]
                    dstv = didx[bj, sl]
                    s1 = plsc.load_gather(s_v, [srcv * 2])
                    s2 = plsc.load_gather(s_v, [dstv * 2 + 1])
                    x = s1 + s2
                    w = jnp.exp(jnp.maximum(x, x * SLOPE))
                    wbuf[bj, pl.ds(j * L, L)] = w
                    plsc.addupdate_scatter(rowsum_v, [srcv], w)
                return carry2

            lax.fori_loop(0, CHA, b_body, carry)
            pltpu.sync_copy(wbuf, w_hbm.at[wid, pl.ds(ch * CHA, CHA)])
            return carry

        lax.fori_loop(0, NCHA, chunk_body, 0)
        pltpu.sync_copy(rowsum_v, rs_hbm.at[wid])

    pl.run_scoped(phase_a,
                  pltpu.VMEM((2 * NPAD,), jnp.float32),
                  pltpu.VMEM((NPAD,), jnp.float32),
                  pltpu.VMEM((CHA, K), jnp.int32),
                  pltpu.VMEM((CHA, K), jnp.int32),
                  pltpu.VMEM((CHA, K), jnp.float32))

    def phase_b(sidx, didx, w_v, gb0, gb1, rows_v):
        gbuf = (gb0, gb1)
        gsem = (sg0, sg1)

        def scale(gb, bj):
            def scale_body(i, carry3):
                wspl = plsc.load_gather(
                    w_v, [jnp.full((L,), bj, jnp.int32),
                          jnp.full((L,), i, jnp.int32)])
                for g in range(D // 32):
                    v = gb[i, pl.ds(g * L, L)]
                    vb = plsc.bitcast(v, jnp.bfloat16)
                    av, bv = plsc.unpack(
                        vb, format=plsc.PackFormat.INTERLEAVED)
                    rows_v[i, pl.ds(32 * g, L)] = av * wspl
                    rows_v[i, pl.ds(32 * g + L, L)] = bv * wspl
                return carry3

            lax.fori_loop(0, K, scale_body, 0)

        def chunk_body(ch, carry):
            pltpu.sync_copy(src_hbm.at[wid, pl.ds(ch * CHB, CHB)], sidx)
            pltpu.sync_copy(dst_hbm.at[wid, pl.ds(ch * CHB, CHB)], didx)
            pltpu.sync_copy(w_hbm.at[wid, pl.ds(ch * CHB, CHB)], w_v)
            pltpu.async_copy(emb_hbm.at[didx.at[0]], gbuf[0], gsem[0])
            for bj in range(CHB):
                p = bj % 2
                q = 1 - p
                pltpu.make_async_copy(
                    emb_hbm.at[didx.at[bj]], gbuf[p], gsem[p]).wait()
                if bj + 1 < CHB:
                    pltpu.async_copy(
                        emb_hbm.at[didx.at[bj + 1]], gbuf[q], gsem[q])
                if bj >= 1:
                    pltpu.make_async_copy(
                        rows_v, acc_sh.at[sidx.at[bj - 1]], ss0).wait()
                scale(gbuf[p], bj)
                pltpu.async_copy(
                    rows_v, acc_sh.at[sidx.at[bj]], ss0, add=True)
            pltpu.make_async_copy(
                rows_v, acc_sh.at[sidx.at[CHB - 1]], ss0).wait()
            return carry

        lax.fori_loop(0, NCHB, chunk_body, 0)

    pl.run_scoped(phase_b,
                  pltpu.VMEM((CHB, K), jnp.int32),
                  pltpu.VMEM((CHB, K), jnp.int32),
                  pltpu.VMEM((CHB, K), jnp.float32),
                  pltpu.VMEM((K, D // 2), jnp.float32),
                  pltpu.VMEM((K, D // 2), jnp.float32),
                  pltpu.VMEM((K, D), jnp.float32))

    plsc.subcore_barrier()
    pltpu.sync_copy(acc_sh.at[pl.ds(sid * RPS, RPS)],
                    out_hbm.at[c, pl.ds(sid * RPS, RPS)])


def _combine_body(p, rs, o):
    tot = p[0] + p[1]
    rsum = jnp.sum(rs[...], axis=1, keepdims=True)
    rsum = jnp.where(rsum == 0.0, 1.0, rsum)
    o[...] = tot / rsum


_combine_call = pl.pallas_call(
    _combine_body,
    grid=(N // RBLK,),
    in_specs=[
        pl.BlockSpec((NC, RBLK, D), lambda i: (0, i, 0)),
        pl.BlockSpec((RBLK, NW), lambda i: (i, 0)),
    ],
    out_specs=pl.BlockSpec((RBLK, D), lambda i: (i, 0)),
    out_shape=jax.ShapeDtypeStruct((N, D), jnp.float32),
)


def kernel(feat_table, nodes, edge_index, W, b, a):
    pad = EPAD - E_TOTAL
    nodes32 = nodes.astype(jnp.int32)
    src_all = jnp.concatenate(
        [edge_index[0], nodes32, jnp.full((pad,), N, jnp.int32)])
    dst_all = jnp.concatenate(
        [edge_index[1], nodes32, jnp.zeros((pad,), jnp.int32)])
    src_r = src_all.reshape(NW, EB, K)
    dst_r = dst_all.reshape(NW, EB, K)
    W_p = W[_PERM, :]
    b_p = b[_PERM].reshape(1, D)
    a_p = jnp.concatenate([a[:D][_PERM], a[D:][_PERM]], axis=0)
    emb16, s = _emb_call(feat_table, W_p, b_p, a_p)
    packed = lax.bitcast_convert_type(
        emb16.reshape(N, D // 2, 2), jnp.float32)
    s_pad = jnp.pad(s, ((0, NPAD - N), (0, 0))).reshape(-1)
    zeros = jnp.zeros((RPS, D), jnp.float32)
    zrs = jnp.zeros((NPAD,), jnp.float32)
    partial, _, rs = _agg(src_r, dst_r, s_pad, packed, zeros, zrs)
    return _combine_call(partial, rs.T)


# speedup vs baseline: 1.4232x; 1.3163x over previous
import functools

import jax
import jax.numpy as jnp
from jax import lax
from jax.experimental import pallas as pl
from jax.experimental.pallas import tpu as pltpu
from jax.experimental.pallas import tpu_sc as plsc

N = 10000
D = 128
E = 320000
SLOPE = 0.1
NC, NS, L = 2, 16, 16
NW = NC * NS
K = 128
EBT = E // K
EB = EBT // NW
XROW = NW * EB
NXB = EBT - XROW
RPS = N // NS
CAB = 13
NCA = EB // CAB
CB = 13
NCB = EB // CB
RBLK = 1000


def _pack_bf16(emb):
    e4 = lax.bitcast_convert_type(emb, jnp.int32).reshape(emb.shape[0], D // 32, 32)
    lo = e4[:, :, :L]
    hi = e4[:, :, L:]
    word = jnp.bitwise_or(
        jnp.bitwise_and(lax.shift_right_arithmetic(lo + 0x8000, 16), 0xFFFF),
        jnp.bitwise_and(hi + 0x8000, jnp.int32(-65536)))
    return lax.bitcast_convert_type(word, jnp.float32).reshape(emb.shape[0], D // 2)


def _emb_body(feat, w, b, a, packed, s, embf):
    emb = lax.dot_general(feat[...], w[...], (((1,), (1,)), ((), ())),
                          preferred_element_type=jnp.float32) + b[...]
    embf[...] = emb
    packed[...] = _pack_bf16(emb)
    s1 = lax.dot_general(emb, a[:D, :], (((1,), (0,)), ((), ())),
                         preferred_element_type=jnp.float32)
    s2 = lax.dot_general(emb, a[D:, :], (((1,), (0,)), ((), ())),
                         preferred_element_type=jnp.float32)
    s[...] = jnp.concatenate([s1, s2], axis=1)


_emb_call = pl.pallas_call(
    _emb_body,
    grid=(N // RBLK,),
    in_specs=[
        pl.BlockSpec((RBLK, D), lambda i: (i, 0)),
        pl.BlockSpec((D, D), lambda i: (0, 0)),
        pl.BlockSpec((1, D), lambda i: (0, 0)),
        pl.BlockSpec((2 * D, 1), lambda i: (0, 0)),
    ],
    out_specs=[
        pl.BlockSpec((RBLK, D // 2), lambda i: (i, 0)),
        pl.BlockSpec((RBLK, 2), lambda i: (i, 0)),
        pl.BlockSpec((RBLK, D), lambda i: (i, 0)),
    ],
    out_shape=[
        jax.ShapeDtypeStruct((N, D // 2), jnp.float32),
        jax.ShapeDtypeStruct((N, 2), jnp.float32),
        jax.ShapeDtypeStruct((N, D), jnp.float32),
    ],
)


_mesh = plsc.VectorSubcoreMesh(core_axis_name="c", subcore_axis_name="s")


@functools.partial(
    pl.kernel,
    out_type=[
        jax.ShapeDtypeStruct((NC, N, D), jnp.float32),
        jax.ShapeDtypeStruct((EBT, K), jnp.float32),
        jax.ShapeDtypeStruct((NW, N), jnp.float32),
    ],
    mesh=_mesh,
    compiler_params=pltpu.CompilerParams(needs_layout_passes=False,
                                         use_tc_tiling_on_sc=False),
    scratch_types=[
        pltpu.VMEM_SHARED((N, D), jnp.float32),
        pltpu.SemaphoreType.DMA,
        pltpu.SemaphoreType.DMA,
        pltpu.SemaphoreType.DMA,
    ],
)
def _agg(src_hbm, dst_hbm, s_hbm, emb_hbm, zeros_hbm, zrs_hbm, out_hbm,
         w_hbm, rs_hbm, acc_sh, sg0, sg1, ss0):
    c = lax.axis_index("c")
    sid = lax.axis_index("s")
    wid = sid * NC + c
    base = wid * EB
    xrow = XROW + wid

    pltpu.sync_copy(zeros_hbm, acc_sh.at[pl.ds(sid * RPS, RPS)])
    plsc.subcore_barrier()

    def phase_a(s_v, rowsum_v, sidx, didx, wbuf):
        pltpu.sync_copy(s_hbm, s_v)
        pltpu.sync_copy(zrs_hbm, rowsum_v)

        def batch_body(bj, carry2):
            for j in range(K // L):
                sl = pl.ds(j * L, L)
                srcv = sidx[bj, sl]
                dstv = didx[bj, sl]
                s1 = plsc.load_gather(s_v, [srcv * 2])
                s2 = plsc.load_gather(s_v, [dstv * 2 + 1])
                x = s1 + s2
                w = jnp.exp(jnp.maximum(x, x * SLOPE))
                wbuf[bj, sl] = w
                plsc.addupdate_scatter(rowsum_v, [srcv], w)
            return carry2

        def chunk_body(ch, carry):
            row = base + ch * CAB
            pltpu.sync_copy(src_hbm.at[pl.ds(row, CAB)], sidx)
            pltpu.sync_copy(dst_hbm.at[pl.ds(row, CAB)], didx)
            lax.fori_loop(0, CAB, batch_body, 0)
            pltpu.sync_copy(wbuf, w_hbm.at[pl.ds(row, CAB)])
            return carry

        lax.fori_loop(0, NCA, chunk_body, 0)

        @pl.when(wid < NXB)
        def _():
            pltpu.sync_copy(src_hbm.at[pl.ds(xrow, 1)], sidx.at[pl.ds(0, 1)])
            pltpu.sync_copy(dst_hbm.at[pl.ds(xrow, 1)], didx.at[pl.ds(0, 1)])
            batch_body(0, 0)
            pltpu.sync_copy(wbuf.at[pl.ds(0, 1)], w_hbm.at[pl.ds(xrow, 1)])

        pltpu.sync_copy(rowsum_v, rs_hbm.at[wid])

    pl.run_scoped(phase_a,
                  pltpu.VMEM((2 * N,), jnp.float32),
                  pltpu.VMEM((N,), jnp.float32),
                  pltpu.VMEM((CAB, K), jnp.int32),
                  pltpu.VMEM((CAB, K), jnp.int32),
                  pltpu.VMEM((CAB, K), jnp.float32))

    def phase_b(sidx, didx, w_v, gb0, gb1, rows_v):
        gbuf = (gb0, gb1)
        gsem = (sg0, sg1)

        def scale(gb, bj):
            def scale_body(i, carry3):
                wspl = plsc.load_gather(
                    w_v, [jnp.full((L,), bj, jnp.int32),
                          jnp.full((L,), i, jnp.int32)])
                for g in range(D // 32):
                    v = gb[i, pl.ds(g * L, L)]
                    vb = plsc.bitcast(v, jnp.bfloat16)
                    av, bv = plsc.unpack(
                        vb, format=plsc.PackFormat.INTERLEAVED)
                    rows_v[i, pl.ds(32 * g, L)] = av * wspl
                    rows_v[i, pl.ds(32 * g + L, L)] = bv * wspl
                return carry3

            lax.fori_loop(0, K, scale_body, 0)

        def chunk_body(ch, carry):
            row = base + ch * CB
            pltpu.sync_copy(src_hbm.at[pl.ds(row, CB)], sidx)
            pltpu.sync_copy(dst_hbm.at[pl.ds(row, CB)], didx)
            pltpu.sync_copy(w_hbm.at[pl.ds(row, CB)], w_v)
            pltpu.async_copy(emb_hbm.at[didx.at[0]], gbuf[0], gsem[0])
            for bj in range(CB):
                p = bj % 2
                q = 1 - p
                pltpu.make_async_copy(
                    emb_hbm.at[didx.at[bj]], gbuf[p], gsem[p]).wait()
                if bj + 1 < CB:
                    pltpu.async_copy(
                        emb_hbm.at[didx.at[bj + 1]], gbuf[q], gsem[q])
                if bj >= 1:
                    pltpu.make_async_copy(
                        rows_v, acc_sh.at[sidx.at[bj - 1]], ss0).wait()
                scale(gbuf[p], bj)
                pltpu.async_copy(
                    rows_v, acc_sh.at[sidx.at[bj]], ss0, add=True)
            pltpu.make_async_copy(
                rows_v, acc_sh.at[sidx.at[CB - 1]], ss0).wait()
            return carry

        lax.fori_loop(0, NCB, chunk_body, 0)

        @pl.when(wid < NXB)
        def _():
            pltpu.sync_copy(src_hbm.at[pl.ds(xrow, 1)], sidx.at[pl.ds(0, 1)])
            pltpu.sync_copy(dst_hbm.at[pl.ds(xrow, 1)], didx.at[pl.ds(0, 1)])
            pltpu.sync_copy(w_hbm.at[pl.ds(xrow, 1)], w_v.at[pl.ds(0, 1)])
            pltpu.async_copy(emb_hbm.at[didx.at[0]], gb0, sg0)
            pltpu.make_async_copy(emb_hbm.at[didx.at[0]], gb0, sg0).wait()
            scale(gb0, 0)
            pltpu.async_copy(rows_v, acc_sh.at[sidx.at[0]], ss0, add=True)
            pltpu.make_async_copy(rows_v, acc_sh.at[sidx.at[0]], ss0).wait()

    pl.run_scoped(phase_b,
                  pltpu.VMEM((CB, K), jnp.int32),
                  pltpu.VMEM((CB, K), jnp.int32),
                  pltpu.VMEM((CB, K), jnp.float32),
                  pltpu.VMEM((K, D // 2), jnp.float32),
                  pltpu.VMEM((K, D // 2), jnp.float32),
                  pltpu.VMEM((K, D), jnp.float32))

    plsc.subcore_barrier()
    pltpu.sync_copy(acc_sh.at[pl.ds(sid * RPS, RPS)],
                    out_hbm.at[c, pl.ds(sid * RPS, RPS)])


def _combine_body(p, rs, s, embf, o):
    x = s[:, 0:1] + s[:, 1:2]
    wself = jnp.exp(jnp.maximum(x, x * SLOPE))
    tot = p[0] + p[1] + wself * embf[...]
    rsum = jnp.sum(rs[...], axis=1, keepdims=True) + wself
    rsum = jnp.where(rsum == 0.0, 1.0, rsum)
    o[...] = tot / rsum


_combine_call = pl.pallas_call(
    _combine_body,
    grid=(N // RBLK,),
    in_specs=[
        pl.BlockSpec((NC, RBLK, D), lambda i: (0, i, 0)),
        pl.BlockSpec((RBLK, NW), lambda i: (i, 0)),
        pl.BlockSpec((RBLK, 2), lambda i: (i, 0)),
        pl.BlockSpec((RBLK, D), lambda i: (i, 0)),
    ],
    out_specs=pl.BlockSpec((RBLK, D), lambda i: (i, 0)),
    out_shape=jax.ShapeDtypeStruct((N, D), jnp.float32),
)


def kernel(feat_table, nodes, edge_index, W, b, a):
    del nodes
    src = edge_index[0].reshape(EBT, K)
    dst = edge_index[1].reshape(EBT, K)
    packed, s, embf = _emb_call(feat_table, W, b.reshape(1, D), a)
    zeros = jnp.zeros((RPS, D), jnp.float32)
    zrs = jnp.zeros((N,), jnp.float32)
    partial, _, rs = _agg(src, dst, s.reshape(-1), packed, zeros, zrs)
    return _combine_call(partial, rs.T, s, embf)


# speedup vs baseline: 1.4716x; 1.0340x over previous
import functools

import jax
import jax.numpy as jnp
from jax import lax
from jax.experimental import pallas as pl
from jax.experimental.pallas import tpu as pltpu
from jax.experimental.pallas import tpu_sc as plsc

N = 10000
D = 128
E = 320000
SLOPE = 0.1
NC, NS, L = 2, 16, 16
NW = NC * NS
K = 128
EBT = E // K
EB = EBT // NW
XROW = NW * EB
NXB = EBT - XROW
RPS = N // NS
CAB = 13
NCA = EB // CAB
CB = 13
NCB = EB // CB
RBLK = 1000


def _emb_body(feat, w, b, a, emb_o, s):
    emb = lax.dot_general(feat[...], w[...], (((1,), (1,)), ((), ())),
                          preferred_element_type=jnp.float32) + b[...]
    emb_o[...] = emb
    s1 = lax.dot_general(emb, a[:D, :], (((1,), (0,)), ((), ())),
                         preferred_element_type=jnp.float32)
    s2 = lax.dot_general(emb, a[D:, :], (((1,), (0,)), ((), ())),
                         preferred_element_type=jnp.float32)
    s[...] = jnp.concatenate([s1, s2], axis=1)


_emb_call = pl.pallas_call(
    _emb_body,
    grid=(N // RBLK,),
    in_specs=[
        pl.BlockSpec((RBLK, D), lambda i: (i, 0)),
        pl.BlockSpec((D, D), lambda i: (0, 0)),
        pl.BlockSpec((1, D), lambda i: (0, 0)),
        pl.BlockSpec((2 * D, 1), lambda i: (0, 0)),
    ],
    out_specs=[
        pl.BlockSpec((RBLK, D), lambda i: (i, 0)),
        pl.BlockSpec((RBLK, 2), lambda i: (i, 0)),
    ],
    out_shape=[
        jax.ShapeDtypeStruct((N, D), jnp.float32),
        jax.ShapeDtypeStruct((N, 2), jnp.float32),
    ],
)


_mesh = plsc.VectorSubcoreMesh(core_axis_name="c", subcore_axis_name="s")


@functools.partial(
    pl.kernel,
    out_type=[
        jax.ShapeDtypeStruct((NC, N, D), jnp.float32),
        jax.ShapeDtypeStruct((EBT, K), jnp.float32),
        jax.ShapeDtypeStruct((NW, N), jnp.float32),
    ],
    mesh=_mesh,
    compiler_params=pltpu.CompilerParams(needs_layout_passes=False,
                                         use_tc_tiling_on_sc=False),
    scratch_types=[
        pltpu.VMEM_SHARED((N, D), jnp.float32),
        pltpu.SemaphoreType.DMA,
        pltpu.SemaphoreType.DMA,
        pltpu.SemaphoreType.DMA,
        pltpu.SemaphoreType.DMA,
    ],
)
def _agg(edges_hbm, s_hbm, emb_hbm, zeros_hbm, zrs_hbm, out_hbm,
         w_hbm, rs_hbm, acc_sh, sg0, sg1, ss0, ss1):
    c = lax.axis_index("c")
    sid = lax.axis_index("s")
    wid = sid * NC + c
    base = wid * EB
    xrow = XROW + wid

    pltpu.sync_copy(zeros_hbm, acc_sh.at[pl.ds(sid * RPS, RPS)])
    plsc.subcore_barrier()

    def phase_a(s_v, rowsum_v, sidx, didx, wbuf):
        pltpu.sync_copy(s_hbm, s_v)
        pltpu.sync_copy(zrs_hbm, rowsum_v)

        def batch_body(bj, carry2):
            for j in range(K // L):
                sl = pl.ds(j * L, L)
                srcv = sidx[bj, sl]
                dstv = didx[bj, sl]
                s1 = plsc.load_gather(s_v, [srcv * 2])
                s2 = plsc.load_gather(s_v, [dstv * 2 + 1])
                x = s1 + s2
                w = jnp.exp(jnp.maximum(x, x * SLOPE))
                wbuf[bj, sl] = w
                plsc.addupdate_scatter(rowsum_v, [srcv], w)
            return carry2

        def chunk_body(ch, carry):
            row = base + ch * CAB
            pltpu.sync_copy(edges_hbm.at[0, pl.ds(row, CAB)], sidx)
            pltpu.sync_copy(edges_hbm.at[1, pl.ds(row, CAB)], didx)
            lax.fori_loop(0, CAB, batch_body, 0)
            pltpu.sync_copy(wbuf, w_hbm.at[pl.ds(row, CAB)])
            return carry

        lax.fori_loop(0, NCA, chunk_body, 0)

        @pl.when(wid < NXB)
        def _():
            pltpu.sync_copy(edges_hbm.at[0, pl.ds(xrow, 1)],
                            sidx.at[pl.ds(0, 1)])
            pltpu.sync_copy(edges_hbm.at[1, pl.ds(xrow, 1)],
                            didx.at[pl.ds(0, 1)])
            batch_body(0, 0)
            pltpu.sync_copy(wbuf.at[pl.ds(0, 1)], w_hbm.at[pl.ds(xrow, 1)])

        pltpu.sync_copy(rowsum_v, rs_hbm.at[wid])

    pl.run_scoped(phase_a,
                  pltpu.VMEM((2 * N,), jnp.float32),
                  pltpu.VMEM((N,), jnp.float32),
                  pltpu.VMEM((CAB, K), jnp.int32),
                  pltpu.VMEM((CAB, K), jnp.int32),
                  pltpu.VMEM((CAB, K), jnp.float32))

    def phase_b(sidx, didx, w_v, gb0, gb1):
        gbuf = (gb0, gb1)
        gsem = (sg0, sg1)
        ssem = (ss0, ss1)

        def scale(gb, bj):
            def scale_body(i, carry3):
                wspl = plsc.load_gather(
                    w_v, [jnp.full((L,), bj, jnp.int32),
                          jnp.full((L,), i, jnp.int32)])
                for g in range(D // L):
                    sl = pl.ds(g * L, L)
                    gb[i, sl] = gb[i, sl] * wspl
                return carry3

            lax.fori_loop(0, K, scale_body, 0)

        def chunk_body(ch, carry):
            row = base + ch * CB
            pltpu.sync_copy(edges_hbm.at[0, pl.ds(row, CB)], sidx)
            pltpu.sync_copy(edges_hbm.at[1, pl.ds(row, CB)], didx)
            pltpu.sync_copy(w_hbm.at[pl.ds(row, CB)], w_v)
            pltpu.async_copy(emb_hbm.at[didx.at[0]], gbuf[0], gsem[0])
            for bj in range(CB):
                p = bj % 2
                q = 1 - p
                pltpu.make_async_copy(
                    emb_hbm.at[didx.at[bj]], gbuf[p], gsem[p]).wait()
                if bj + 1 < CB:
                    if bj >= 1:
                        pltpu.make_async_copy(
                            gbuf[q], acc_sh.at[sidx.at[bj - 1]],
                            ssem[q]).wait()
                    pltpu.async_copy(
                        emb_hbm.at[didx.at[bj + 1]], gbuf[q], gsem[q])
                scale(gbuf[p], bj)
                pltpu.async_copy(
                    gbuf[p], acc_sh.at[sidx.at[bj]], ssem[p], add=True)
            pltpu.make_async_copy(
                gbuf[0], acc_sh.at[sidx.at[CB - 1]], ssem[0]).wait()
            pltpu.make_async_copy(
                gbuf[1], acc_sh.at[sidx.at[CB - 2]], ssem[1]).wait()
            return carry

        lax.fori_loop(0, NCB, chunk_body, 0)

        @pl.when(wid < NXB)
        def _():
            pltpu.sync_copy(edges_hbm.at[0, pl.ds(xrow, 1)],
                            sidx.at[pl.ds(0, 1)])
            pltpu.sync_copy(edges_hbm.at[1, pl.ds(xrow, 1)],
                            didx.at[pl.ds(0, 1)])
            pltpu.sync_copy(w_hbm.at[pl.ds(xrow, 1)], w_v.at[pl.ds(0, 1)])
            pltpu.async_copy(emb_hbm.at[didx.at[0]], gb0, sg0)
            pltpu.make_async_copy(emb_hbm.at[didx.at[0]], gb0, sg0).wait()
            scale(gb0, 0)
            pltpu.async_copy(gb0, acc_sh.at[sidx.at[0]], ss0, add=True)
            pltpu.make_async_copy(gb0, acc_sh.at[sidx.at[0]], ss0).wait()

    pl.run_scoped(phase_b,
                  pltpu.VMEM((CB, K), jnp.int32),
                  pltpu.VMEM((CB, K), jnp.int32),
                  pltpu.VMEM((CB, K), jnp.float32),
                  pltpu.VMEM((K, D), jnp.float32),
                  pltpu.VMEM((K, D), jnp.float32))

    plsc.subcore_barrier()
    pltpu.sync_copy(acc_sh.at[pl.ds(sid * RPS, RPS)],
                    out_hbm.at[c, pl.ds(sid * RPS, RPS)])


def _combine_body(p, rs, s, embf, o):
    x = s[:, 0:1] + s[:, 1:2]
    wself = jnp.exp(jnp.maximum(x, x * SLOPE))
    tot = p[0] + p[1] + wself * embf[...]
    rsum = jnp.sum(rs[...], axis=1, keepdims=True) + wself
    rsum = jnp.where(rsum == 0.0, 1.0, rsum)
    o[...] = tot / rsum


_combine_call = pl.pallas_call(
    _combine_body,
    grid=(N // RBLK,),
    in_specs=[
        pl.BlockSpec((NC, RBLK, D), lambda i: (0, i, 0)),
        pl.BlockSpec((RBLK, NW), lambda i: (i, 0)),
        pl.BlockSpec((RBLK, 2), lambda i: (i, 0)),
        pl.BlockSpec((RBLK, D), lambda i: (i, 0)),
    ],
    out_specs=pl.BlockSpec((RBLK, D), lambda i: (i, 0)),
    out_shape=jax.ShapeDtypeStruct((N, D), jnp.float32),
)


def kernel(feat_table, nodes, edge_index, W, b, a):
    del nodes
    edges3 = edge_index.reshape(2, EBT, K)
    emb, s = _emb_call(feat_table, W, b.reshape(1, D), a)
    zeros = jnp.zeros((RPS, D), jnp.float32)
    zrs = jnp.zeros((N,), jnp.float32)
    partial, _, rs = _agg(edges3, s.reshape(-1), emb, zeros, zrs)
    return _combine_call(partial, rs.T, s, emb)
